# Initial kernel scaffold; baseline (speedup 1.0000x reference)
import functools

import jax
import jax.numpy as jnp
from jax import lax
from jax.experimental import pallas as pl
from jax.experimental.pallas import tpu as pltpu
from jax.experimental.pallas import tpu_sc as plsc

N = 10000
DIM_IN = 128
DIM_OUT = 10
H = 8
HID = 8
E = 320000

NPAD = 10240
NW = 32
EB = 128
NB = (E + N + NW * EB - 1) // (NW * EB)
EPAD = NW * NB * EB
ROWS_PER_TILE = NPAD // 16
CHUNKS_PER_TILE = ROWS_PER_TILE // EB


def _leaky(x):
    return jnp.where(x > 0, x, 0.2 * x)


def _prep1_body(x_ref, w_ref, tab_s, tab_d, m_ref):
    prod = jnp.dot(x_ref[...], w_ref[...], preferred_element_type=jnp.float32)
    tab_s[...] = prod[:, :80]
    tab_d[...] = prod[:, 80:96]
    ms = jnp.max(prod[:, 64:80], axis=0) + jnp.max(prod[:, 80:96], axis=0)
    m_ref[...] = _leaky(ms).reshape(1, 16)


def _prep2_body(p_ref, b1_ref, w_ref, tab_s, tab_d, m_ref):
    s = p_ref[0] + p_ref[1]
    den = s[:, 64:72]
    den64 = jnp.concatenate([den] * 8, axis=1)
    feat = s[:, :64] / (den64 + 1e-16) + b1_ref[...]
    eluf = jnp.where(feat > 0, feat, jnp.exp(feat) - 1.0)
    prod = jnp.dot(eluf, w_ref[...], preferred_element_type=jnp.float32)
    tab_s[...] = prod[:, :96]
    tab_d[...] = prod[:, 96:112]
    ms = jnp.max(prod[:, 80:96], axis=0) + jnp.max(prod[:, 96:112], axis=0)
    m_ref[...] = _leaky(ms).reshape(1, 16)


def _final_body(p_ref, b2_ref, a_ref, o_ref):
    s = p_ref[0] + p_ref[1]
    den = s[:, 80:88]
    den80 = jnp.concatenate([den] * 10, axis=1)
    feat = s[:, :80] / (den80 + 1e-16)
    g = jnp.dot(feat, a_ref[...], preferred_element_type=jnp.float32)
    g = g + b2_ref[...]
    mx = jnp.max(g, axis=1, keepdims=True)
    z = g - mx
    lse = jnp.log(jnp.sum(jnp.exp(z), axis=1, keepdims=True))
    o_ref[...] = z - lse


_prep1 = pl.pallas_call(
    _prep1_body,
    out_shape=[
        jax.ShapeDtypeStruct((NPAD, 80), jnp.float32),
        jax.ShapeDtypeStruct((NPAD, 16), jnp.float32),
        jax.ShapeDtypeStruct((1, 16), jnp.float32),
    ],
)

_prep2 = pl.pallas_call(
    _prep2_body,
    out_shape=[
        jax.ShapeDtypeStruct((NPAD, 96), jnp.float32),
        jax.ShapeDtypeStruct((NPAD, 16), jnp.float32),
        jax.ShapeDtypeStruct((1, 16), jnp.float32),
    ],
)

_final = pl.pallas_call(
    _final_body,
    out_shape=jax.ShapeDtypeStruct((NPAD, 10), jnp.float32),
)


def _make_edge_kernel(ws, wacc):
    wh = ws - 16
    mesh = plsc.VectorSubcoreMesh(core_axis_name="c", subcore_axis_name="s")

    def body(tab_s, tab_d, sidx_h, didx_h, m_h, zer_h, out_h,
             sidx_v, didx_v, srows, drows, msg, mvec,
             accum, sem_s, sem_d, sem_w):
        cid = lax.axis_index("c")
        sid = lax.axis_index("s")
        wid = sid * 2 + cid

        pltpu.sync_copy(sidx_h.at[wid], sidx_v)
        pltpu.sync_copy(didx_h.at[wid], didx_v)
        pltpu.sync_copy(m_h, mvec)

        base = sid * ROWS_PER_TILE
        for k in range(CHUNKS_PER_TILE):
            pltpu.sync_copy(zer_h, accum.at[pl.ds(base + k * EB, EB)])
        plsc.subcore_barrier()

        mval = mvec[...]

        def start_gathers(g, buf):
            pltpu.async_copy(tab_s.at[sidx_v.at[g]], srows.at[buf], sem_s.at[buf])
            pltpu.async_copy(tab_d.at[didx_v.at[g]], drows.at[buf], sem_d.at[buf])

        def wait_gathers(g, buf):
            pltpu.make_async_copy(tab_s.at[sidx_v.at[g]], srows.at[buf],
                                  sem_s.at[buf]).wait()
            pltpu.make_async_copy(tab_d.at[didx_v.at[g]], drows.at[buf],
                                  sem_d.at[buf]).wait()

        def wait_scatter(g, buf):
            pltpu.make_async_copy(msg.at[buf], accum.at[didx_v.at[g]],
                                  sem_w.at[buf]).wait()

        start_gathers(0, 0)

        def outer(g, carry):
            buf = lax.rem(g, 2)

            @pl.when(g + 1 < NB)
            def _():
                start_gathers(g + 1, 1 - buf)

            @pl.when(g >= 2)
            def _():
                wait_scatter(g - 2, buf)

            wait_gathers(g, buf)

            def inner(i, c):
                a = srows[buf, i, pl.ds(wh, 16)]
                b = drows[buf, i, :]
                p = jnp.exp(_leaky(a + b) - mval)
                for j in range(wh // 16):
                    sl = pl.ds(j * 16, 16)
                    msg[buf, i, sl] = srows[buf, i, sl] * p
                msg[buf, i, pl.ds(wh, 16)] = p
                return c

            lax.fori_loop(0, EB, inner, 0, unroll=2)

            pltpu.make_async_copy(msg.at[buf], accum.at[didx_v.at[g]],
                                  sem_w.at[buf]).start(add=True)
            return carry

        lax.fori_loop(0, NB, outer, 0)
        wait_scatter(NB - 2, (NB - 2) % 2)
        wait_scatter(NB - 1, (NB - 1) % 2)
        plsc.subcore_barrier()

        for k in range(CHUNKS_PER_TILE):
            sl = pl.ds(base + k * EB, EB)
            pltpu.sync_copy(accum.at[sl], out_h.at[cid].at[sl])

    return pl.kernel(
        body,
        out_type=jax.ShapeDtypeStruct((2, NPAD, wacc), jnp.float32),
        mesh=mesh,
        scratch_types=[
            pltpu.VMEM((NB, EB), jnp.int32),
            pltpu.VMEM((NB, EB), jnp.int32),
            pltpu.VMEM((2, EB, ws), jnp.float32),
            pltpu.VMEM((2, EB, 16), jnp.float32),
            pltpu.VMEM((2, EB, wacc), jnp.float32),
            pltpu.VMEM((16,), jnp.float32),
            pltpu.VMEM_SHARED((NPAD, wacc), jnp.float32),
            pltpu.SemaphoreType.DMA((2,)),
            pltpu.SemaphoreType.DMA((2,)),
            pltpu.SemaphoreType.DMA((2,)),
        ],
    )


_edge1 = _make_edge_kernel(80, 80)
_edge2 = _make_edge_kernel(96, 96)


def kernel(x, edge_index, W1, a_src1, a_dst1, b1, W2, a_src2, a_dst2, b2):
    f32 = jnp.float32

    ch1 = jnp.arange(64)
    perm1 = (ch1 % 8) * 8 + ch1 // 8
    W1p = W1[:, perm1]
    w_as1 = jnp.einsum("ihc,hc->ih", W1.reshape(DIM_IN, H, HID), a_src1)
    w_ad1 = jnp.einsum("ihc,hc->ih", W1.reshape(DIM_IN, H, HID), a_dst1)
    Wcat1 = jnp.concatenate([W1p, w_as1, w_as1, w_ad1, w_ad1], axis=1)

    ch2 = jnp.arange(80)
    perm2c = (ch2 % 8) * 10 + ch2 // 8
    W2p = W2[perm1][:, perm2c]
    w_as2 = jnp.einsum("ihc,hc->ih", W2[perm1].reshape(64, H, DIM_OUT), a_src2)
    w_ad2 = jnp.einsum("ihc,hc->ih", W2[perm1].reshape(64, H, DIM_OUT), a_dst2)
    Wcat2 = jnp.concatenate([W2p, w_as2, w_as2, w_ad2, w_ad2], axis=1)

    b1c = b1[perm1].reshape(1, 64)
    amean = jnp.zeros((80, 10), f32).at[ch2, ch2 // 8].set(0.125)
    b2r = b2.reshape(1, DIM_OUT).astype(f32)

    loop = jnp.arange(N, dtype=jnp.int32)
    pad = jnp.full((EPAD - E - N,), N, jnp.int32)
    src = jnp.concatenate([edge_index[0], loop, pad]).reshape(NW, NB, EB)
    dst = jnp.concatenate([edge_index[1], loop, pad]).reshape(NW, NB, EB)

    xp = jnp.zeros((NPAD, DIM_IN), f32).at[:N].set(x)
    z80 = jnp.zeros((EB, 80), f32)
    z96 = jnp.zeros((EB, 96), f32)

    tab_s1, tab_d1, m1 = _prep1(xp, Wcat1)
    part1 = _edge1(tab_s1, tab_d1, src, dst, m1.reshape(16), z80)

    tab_s2, tab_d2, m2 = _prep2(part1, b1c, Wcat2)
    part2 = _edge2(tab_s2, tab_d2, src, dst, m2.reshape(16), z96)

    out = _final(part2, b2r, amean)
    return out[:N]


# speedup vs baseline: 92.6779x; 92.6779x over previous
import functools

import jax
import jax.numpy as jnp
from jax import lax
from jax.experimental import pallas as pl
from jax.experimental.pallas import tpu as pltpu
from jax.experimental.pallas import tpu_sc as plsc

N = 10000
DIM_IN = 128
DIM_OUT = 10
H = 8
HID = 8
E = 320000

NPAD = 10240
NW = 32
EB = 64
NB = (E + N + NW * EB - 1) // (NW * EB)
EPAD = NW * NB * EB
ROWS_PER_TILE = NPAD // 16
CHUNKS_PER_TILE = ROWS_PER_TILE // EB


def _leaky(x):
    return jnp.where(x > 0, x, 0.2 * x)


def _prep1_body(x_ref, w_ref, tab_s, tab_d, m_ref):
    prod = jnp.dot(x_ref[...], w_ref[...], preferred_element_type=jnp.float32)
    tab_s[...] = prod[:, :80]
    tab_d[...] = prod[:, 80:96]
    ms = jnp.max(prod[:, 64:80], axis=0) + jnp.max(prod[:, 80:96], axis=0)
    m_ref[...] = _leaky(ms).reshape(1, 16)


def _prep2_body(p_ref, b1_ref, w_ref, tab_s, tab_d, m_ref):
    s = p_ref[0] + p_ref[1]
    den = s[:, 64:72]
    den64 = jnp.concatenate([den] * 8, axis=1)
    feat = s[:, :64] / (den64 + 1e-16) + b1_ref[...]
    eluf = jnp.where(feat > 0, feat, jnp.exp(feat) - 1.0)
    prod = jnp.dot(eluf, w_ref[...], preferred_element_type=jnp.float32)
    tab_s[...] = prod[:, :96]
    tab_d[...] = prod[:, 96:112]
    ms = jnp.max(prod[:, 80:96], axis=0) + jnp.max(prod[:, 96:112], axis=0)
    m_ref[...] = _leaky(ms).reshape(1, 16)


def _final_body(p_ref, b2_ref, a_ref, o_ref):
    s = p_ref[0] + p_ref[1]
    den = s[:, 80:88]
    den80 = jnp.concatenate([den] * 10, axis=1)
    feat = s[:, :80] / (den80 + 1e-16)
    g = jnp.dot(feat, a_ref[...], preferred_element_type=jnp.float32)
    g = g + b2_ref[...]
    mx = jnp.max(g, axis=1, keepdims=True)
    z = g - mx
    lse = jnp.log(jnp.sum(jnp.exp(z), axis=1, keepdims=True))
    o_ref[...] = z - lse


_prep1 = pl.pallas_call(
    _prep1_body,
    compiler_params=pltpu.CompilerParams(vmem_limit_bytes=100 * 1024 * 1024),
    out_shape=[
        jax.ShapeDtypeStruct((NPAD, 80), jnp.float32),
        jax.ShapeDtypeStruct((NPAD, 16), jnp.float32),
        jax.ShapeDtypeStruct((1, 16), jnp.float32),
    ],
)

_prep2 = pl.pallas_call(
    _prep2_body,
    compiler_params=pltpu.CompilerParams(vmem_limit_bytes=100 * 1024 * 1024),
    out_shape=[
        jax.ShapeDtypeStruct((NPAD, 96), jnp.float32),
        jax.ShapeDtypeStruct((NPAD, 16), jnp.float32),
        jax.ShapeDtypeStruct((1, 16), jnp.float32),
    ],
)

_final = pl.pallas_call(
    _final_body,
    compiler_params=pltpu.CompilerParams(vmem_limit_bytes=100 * 1024 * 1024),
    out_shape=jax.ShapeDtypeStruct((NPAD, 10), jnp.float32),
)


def _make_edge_kernel(ws, wacc):
    wh = ws - 16
    mesh = plsc.VectorSubcoreMesh(core_axis_name="c", subcore_axis_name="s")

    def body(tab_s, tab_d, sidx_h, didx_h, m_h, zer_h, out_h,
             sidx_v, didx_v, srows, drows, msg, mvec,
             accum, sem_s, sem_d, sem_w):
        cid = lax.axis_index("c")
        sid = lax.axis_index("s")
        wid = sid * 2 + cid

        pltpu.sync_copy(sidx_h.at[wid], sidx_v)
        pltpu.sync_copy(didx_h.at[wid], didx_v)
        pltpu.sync_copy(m_h, mvec)

        base = sid * ROWS_PER_TILE
        for k in range(CHUNKS_PER_TILE):
            pltpu.sync_copy(zer_h, accum.at[pl.ds(base + k * EB, EB)])
        plsc.subcore_barrier()

        mval = mvec[...]

        def start_gathers(g, buf):
            pltpu.async_copy(tab_s.at[sidx_v.at[g]], srows.at[buf], sem_s.at[buf])
            pltpu.async_copy(tab_d.at[didx_v.at[g]], drows.at[buf], sem_d.at[buf])

        def wait_gathers(g, buf):
            pltpu.make_async_copy(tab_s.at[sidx_v.at[g]], srows.at[buf],
                                  sem_s.at[buf]).wait()
            pltpu.make_async_copy(tab_d.at[didx_v.at[g]], drows.at[buf],
                                  sem_d.at[buf]).wait()

        def wait_scatter(g, buf):
            pltpu.make_async_copy(msg.at[buf], accum.at[didx_v.at[g]],
                                  sem_w.at[buf]).wait()

        start_gathers(0, 0)

        def outer(g, carry):
            buf = lax.rem(g, 2)

            @pl.when(g + 1 < NB)
            def _():
                start_gathers(g + 1, 1 - buf)

            @pl.when(g >= 2)
            def _():
                wait_scatter(g - 2, buf)

            wait_gathers(g, buf)

            def inner(i, c):
                a = srows[buf, i, pl.ds(wh, 16)]
                b = drows[buf, i, :]
                p = jnp.exp(_leaky(a + b) - mval)
                for j in range(wh // 16):
                    sl = pl.ds(j * 16, 16)
                    msg[buf, i, sl] = srows[buf, i, sl] * p
                msg[buf, i, pl.ds(wh, 16)] = p
                return c

            lax.fori_loop(0, EB, inner, 0, unroll=2)

            pltpu.make_async_copy(msg.at[buf], accum.at[didx_v.at[g]],
                                  sem_w.at[buf]).start(add=True)
            return carry

        lax.fori_loop(0, NB, outer, 0)
        wait_scatter(NB - 2, (NB - 2) % 2)
        wait_scatter(NB - 1, (NB - 1) % 2)
        plsc.subcore_barrier()

        for k in range(CHUNKS_PER_TILE):
            sl = pl.ds(base + k * EB, EB)
            pltpu.sync_copy(accum.at[sl], out_h.at[cid].at[sl])

    return pl.kernel(
        body,
        out_type=jax.ShapeDtypeStruct((2, NPAD, wacc), jnp.float32),
        mesh=mesh,
        compiler_params=pltpu.CompilerParams(use_tc_tiling_on_sc=False),
        scratch_types=[
            pltpu.VMEM((NB, EB), jnp.int32),
            pltpu.VMEM((NB, EB), jnp.int32),
            pltpu.VMEM((2, EB, ws), jnp.float32),
            pltpu.VMEM((2, EB, 16), jnp.float32),
            pltpu.VMEM((2, EB, wacc), jnp.float32),
            pltpu.VMEM((16,), jnp.float32),
            pltpu.VMEM_SHARED((NPAD, wacc), jnp.float32),
            pltpu.SemaphoreType.DMA((2,)),
            pltpu.SemaphoreType.DMA((2,)),
            pltpu.SemaphoreType.DMA((2,)),
        ],
    )


_edge1 = _make_edge_kernel(80, 80)
_edge2 = _make_edge_kernel(96, 96)


def kernel(x, edge_index, W1, a_src1, a_dst1, b1, W2, a_src2, a_dst2, b2):
    f32 = jnp.float32

    ch1 = jnp.arange(64)
    perm1 = (ch1 % 8) * 8 + ch1 // 8
    W1p = W1[:, perm1]
    w_as1 = jnp.einsum("ihc,hc->ih", W1.reshape(DIM_IN, H, HID), a_src1)
    w_ad1 = jnp.einsum("ihc,hc->ih", W1.reshape(DIM_IN, H, HID), a_dst1)
    Wcat1 = jnp.concatenate([W1p, w_as1, w_as1, w_ad1, w_ad1], axis=1)

    ch2 = jnp.arange(80)
    perm2c = (ch2 % 8) * 10 + ch2 // 8
    W2p = W2[perm1][:, perm2c]
    w_as2 = jnp.einsum("ihc,hc->ih", W2[perm1].reshape(64, H, DIM_OUT), a_src2)
    w_ad2 = jnp.einsum("ihc,hc->ih", W2[perm1].reshape(64, H, DIM_OUT), a_dst2)
    Wcat2 = jnp.concatenate([W2p, w_as2, w_as2, w_ad2, w_ad2], axis=1)

    b1c = b1[perm1].reshape(1, 64)
    amean = jnp.zeros((80, 10), f32).at[ch2, ch2 // 8].set(0.125)
    b2r = b2.reshape(1, DIM_OUT).astype(f32)

    loop = jnp.arange(N, dtype=jnp.int32)
    pad = jnp.full((EPAD - E - N,), N, jnp.int32)
    src = jnp.concatenate([edge_index[0], loop, pad]).reshape(NW, NB, EB)
    dst = jnp.concatenate([edge_index[1], loop, pad]).reshape(NW, NB, EB)

    xp = jnp.zeros((NPAD, DIM_IN), f32).at[:N].set(x)
    z80 = jnp.zeros((EB, 80), f32)
    z96 = jnp.zeros((EB, 96), f32)

    tab_s1, tab_d1, m1 = _prep1(xp, Wcat1)
    part1 = _edge1(tab_s1, tab_d1, src, dst, m1.reshape(16), z80)

    tab_s2, tab_d2, m2 = _prep2(part1, b1c, Wcat2)
    part2 = _edge2(tab_s2, tab_d2, src, dst, m2.reshape(16), z96)

    out = _final(part2, b2r, amean)
    return out[:N]
